# Initial kernel scaffold; baseline (speedup 1.0000x reference)
import jax
import jax.numpy as jnp
from jax.experimental import pallas as pl


def kernel(x, edge_index, batch, W1, b1, W2, b2, W3, b3, Wl, bl):
    raise NotImplementedError("write your pallas kernel here")


# speedup vs baseline: 27.7433x; 27.7433x over previous
import functools

import jax
import jax.numpy as jnp
from jax import lax
from jax.experimental import pallas as pl
from jax.experimental.pallas import tpu as pltpu
from jax.experimental.pallas import tpu_sc as plsc

N = 10000
E = 320000
D = 128
H = 128
C = 16
G = 64

NC = 2
NS = 16
NW = NC * NS
EPT = E // NW
CHUNK = 40
NCHUNK = EPT // CHUNK
ROWS_A = 632
ROWS_LAST = N - (NS - 1) * ROWS_A
NBUF = 5
PD = 2


def _mesh():
    return plsc.VectorSubcoreMesh(
        core_axis_name="c", subcore_axis_name="s",
        num_cores=NC, num_subcores=NS)


def _sc_params():
    return pltpu.CompilerParams(use_tc_tiling_on_sc=False)


def _striped(fn, s, base=0):
    @pl.when(s < NS - 1)
    def _():
        fn(base + s * ROWS_A, ROWS_A)

    @pl.when(s == NS - 1)
    def _():
        fn(base + (NS - 1) * ROWS_A, ROWS_LAST)


def _sc_hist_body(dst_hbm, zeros_hbm, ones_hbm, out_hbm,
                  acc, didx, ones_v, h0, h1, h2, h3, h4):
    c = lax.axis_index("c")
    s = lax.axis_index("s")
    wid = s * NC + c
    hsem = [h0, h1, h2, h3, h4]

    _striped(lambda off, rows: pltpu.sync_copy(
        zeros_hbm.at[pl.ds(off, rows)], acc.at[pl.ds(off, rows)]), s)
    pltpu.sync_copy(ones_hbm, ones_v)
    pltpu.sync_copy(dst_hbm.at[wid], didx)
    plsc.subcore_barrier()

    @pl.loop(0, NCHUNK // NBUF)
    def _(o):
        for b in range(NBUF):
            j = o * NBUF + b
            pltpu.async_copy(ones_v, acc.at[didx.at[j]], hsem[b], add=True)
        for b in range(NBUF):
            pltpu.make_async_copy(zeros_hbm.at[pl.ds(0, CHUNK)], ones_v,
                                  hsem[b]).wait()

    plsc.subcore_barrier()
    _striped(lambda off, rows: pltpu.sync_copy(
        acc.at[pl.ds(off - c * N, rows)], out_hbm.at[pl.ds(off, rows)]),
        s, base=c * N)


def _sc_hist(dst, zeros_n16, ones_c16):
    k = pl.kernel(
        _sc_hist_body,
        out_type=jax.ShapeDtypeStruct((NC * N, 16), jnp.float32),
        mesh=_mesh(),
        scratch_types=[
            pltpu.VMEM_SHARED((N, 16), jnp.float32),
            pltpu.VMEM((NCHUNK, CHUNK), jnp.int32),
            pltpu.VMEM((CHUNK, 16), jnp.float32),
        ] + [pltpu.SemaphoreType.DMA] * NBUF,
        compiler_params=_sc_params(),
    )
    return k(dst, zeros_n16, ones_c16)


def _sc_agg_body(hp_hbm, src_hbm, dst_hbm, zeros_hbm, out_hbm,
                 acc, sidx, didx, b0, b1, b2, b3, b4,
                 g0, g1, g2, g3, g4, s0, s1, s2, s3, s4):
    c = lax.axis_index("c")
    s = lax.axis_index("s")
    wid = s * NC + c
    bufs = [b0, b1, b2, b3, b4]
    gsem = [g0, g1, g2, g3, g4]
    ssem = [s0, s1, s2, s3, s4]

    _striped(lambda off, rows: pltpu.sync_copy(
        zeros_hbm.at[pl.ds(off, rows)], acc.at[pl.ds(off, rows)]), s)
    pltpu.sync_copy(src_hbm.at[wid], sidx)
    pltpu.sync_copy(dst_hbm.at[wid], didx)
    plsc.subcore_barrier()

    for b in range(PD):
        pltpu.async_copy(hp_hbm.at[sidx.at[b]], bufs[b], gsem[b])

    @pl.loop(0, NCHUNK // NBUF)
    def _(o):
        for b in range(NBUF):
            j = o * NBUF + b
            jg = j + PD
            bg = (b + PD) % NBUF

            @pl.when(jg < NCHUNK)
            def _():
                @pl.when(jg >= NBUF)
                def _():
                    pltpu.make_async_copy(hp_hbm.at[pl.ds(0, CHUNK)],
                                          bufs[bg], ssem[bg]).wait()
                pltpu.async_copy(hp_hbm.at[sidx.at[jg]], bufs[bg], gsem[bg])

            pltpu.make_async_copy(hp_hbm.at[pl.ds(0, CHUNK)], bufs[b],
                                  gsem[b]).wait()
            pltpu.async_copy(bufs[b], acc.at[didx.at[j]], ssem[b], add=True)

    for b in range(NBUF):
        pltpu.make_async_copy(hp_hbm.at[pl.ds(0, CHUNK)], bufs[b],
                              ssem[b]).wait()

    plsc.subcore_barrier()
    _striped(lambda off, rows: pltpu.sync_copy(
        acc.at[pl.ds(off - c * N, rows)], out_hbm.at[pl.ds(off, rows)]),
        s, base=c * N)


def _sc_agg(hp, src, dst, zeros_nd):
    k = pl.kernel(
        _sc_agg_body,
        out_type=jax.ShapeDtypeStruct((NC * N, D), jnp.float32),
        mesh=_mesh(),
        scratch_types=[
            pltpu.VMEM_SHARED((N, D), jnp.float32),
            pltpu.VMEM((NCHUNK, CHUNK), jnp.int32),
            pltpu.VMEM((NCHUNK, CHUNK), jnp.int32),
        ] + [pltpu.VMEM((CHUNK, D), jnp.float32)] * NBUF
          + [pltpu.SemaphoreType.DMA] * (2 * NBUF),
        compiler_params=_sc_params(),
    )
    return k(hp, src, dst, zeros_nd)


def _tc_mm_body(x_ref, w_ref, o_ref):
    o_ref[...] = jnp.dot(x_ref[...], w_ref[...],
                         preferred_element_type=jnp.float32)


def _tc_mm(x, w):
    return pl.pallas_call(
        _tc_mm_body,
        out_shape=jax.ShapeDtypeStruct((x.shape[0], w.shape[1]), jnp.float32),
    )(x, w)


def _tc_scale_body(cnt_ref, h_ref, hp_ref, dis_ref, invd_ref):
    deg = cnt_ref[0] + cnt_ref[1] + 1.0
    deg = deg[:, 0:1]
    dis = lax.rsqrt(deg)
    invd = 1.0 / deg
    dis_ref[...] = dis
    invd_ref[...] = invd
    hp_ref[...] = dis * h_ref[...]


def _tc_scale(cnt, h):
    return pl.pallas_call(
        _tc_scale_body,
        out_shape=[
            jax.ShapeDtypeStruct((N, H), jnp.float32),
            jax.ShapeDtypeStruct((N, 1), jnp.float32),
            jax.ShapeDtypeStruct((N, 1), jnp.float32),
        ],
    )(cnt, h)


def _tc_mid_body(p_ref, h_ref, dis_ref, invd_ref, b_ref, w_ref,
                 hn_ref, hpn_ref):
    dis = dis_ref[...]
    z = (dis * (p_ref[0] + p_ref[1])
         + h_ref[...] * invd_ref[...] + b_ref[...])
    mu = jnp.mean(z, axis=0, keepdims=True)
    zc = z - mu
    var = jnp.mean(zc * zc, axis=0, keepdims=True)
    zn = jnp.maximum(zc * lax.rsqrt(var + 1e-5), 0.0)
    hn = jnp.dot(zn, w_ref[...], preferred_element_type=jnp.float32)
    hn_ref[...] = hn
    hpn_ref[...] = dis * hn


def _tc_mid(p, h, dis, invd, b, w):
    return pl.pallas_call(
        _tc_mid_body,
        out_shape=[
            jax.ShapeDtypeStruct((N, H), jnp.float32),
            jax.ShapeDtypeStruct((N, H), jnp.float32),
        ],
    )(p, h, dis, invd, b, w)


def _tc_final_body(p_ref, h_ref, dis_ref, invd_ref, b_ref, batch_ref,
                   wl_ref, bl_ref, o_ref):
    z = (dis_ref[...] * (p_ref[0] + p_ref[1])
         + h_ref[...] * invd_ref[...] + b_ref[...])
    seg = lax.broadcasted_iota(jnp.int32, (G, N), 0)
    onehot = (seg == batch_ref[...]).astype(jnp.float32)
    sums = jnp.dot(onehot, z, preferred_element_type=jnp.float32)
    cnt = jnp.sum(onehot, axis=1, keepdims=True)
    pooled = sums / jnp.maximum(cnt, 1.0)
    o_ref[...] = jnp.dot(pooled, wl_ref[...],
                         preferred_element_type=jnp.float32) + bl_ref[...]


def _tc_final(p, h, dis, invd, b, batch2d, wl, bl):
    return pl.pallas_call(
        _tc_final_body,
        out_shape=jax.ShapeDtypeStruct((G, C), jnp.float32),
    )(p, h, dis, invd, b, batch2d, wl, bl)


def kernel(x, edge_index, batch, W1, b1, W2, b2, W3, b3, Wl, bl):
    src = edge_index[0].reshape(NW, NCHUNK, CHUNK)
    dst = edge_index[1].reshape(NW, NCHUNK, CHUNK)
    zeros_nd = jnp.zeros((N, D), jnp.float32)
    zeros_n16 = jnp.zeros((N, 16), jnp.float32)
    ones_c16 = jnp.ones((CHUNK, 16), jnp.float32)

    cnt = _sc_hist(dst, zeros_n16, ones_c16).reshape(NC, N, 16)
    h1 = _tc_mm(x, W1)
    hp1, dis, invd = _tc_scale(cnt, h1)

    p1 = _sc_agg(hp1, src, dst, zeros_nd).reshape(NC, N, D)
    h2, hp2 = _tc_mid(p1, h1, dis, invd, b1.reshape(1, H), W2)

    p2 = _sc_agg(hp2, src, dst, zeros_nd).reshape(NC, N, D)
    h3, hp3 = _tc_mid(p2, h2, dis, invd, b2.reshape(1, H), W3)

    p3 = _sc_agg(hp3, src, dst, zeros_nd).reshape(NC, N, D)
    return _tc_final(p3, h3, dis, invd, b3.reshape(1, H),
                     batch.reshape(1, N), Wl, bl.reshape(1, C))
